# baseline (speedup 1.0000x reference)
import functools

import jax
import jax.numpy as jnp
from jax import lax
from jax.experimental import pallas as pl
from jax.experimental.pallas import tpu as pltpu, tpu_sc as plsc

N = 10000
E = 320000
NCL = 5000
NT = 32
TS = 157
CAP = 20000
CAPU = CAP + 32
CAPF = CAP + 512
SENT = (1 << 21) - 1
SLICE = E // NT
W2 = 4000
NWIN = E // W2
FST = 320
FLUSH = 256
ROWB = 16 * 128 * 4

_mesh = plsc.VectorSubcoreMesh(core_axis_name="c", subcore_axis_name="s",
                               num_cores=2, num_subcores=16)
_params = pltpu.CompilerParams(needs_layout_passes=False)


def _wid():
    return lax.axis_index("s") * 2 + lax.axis_index("c")


def _iota():
    return lax.iota(jnp.int32, 16)


@functools.partial(
    pl.kernel,
    out_type=[jax.ShapeDtypeStruct((E,), jnp.int32),
              jax.ShapeDtypeStruct((E,), jnp.int32)],
    mesh=_mesh,
    scratch_types=[pltpu.VMEM((N,), jnp.int32),
                   pltpu.VMEM((SLICE,), jnp.int32),
                   pltpu.VMEM((SLICE,), jnp.int32)],
    compiler_params=_params,
)
def _k1(clu_h, e0_h, e1_h, s_h, d_h, clu_v, ein_v, out_v):
    w = _wid()
    base = w * SLICE
    pltpu.sync_copy(clu_h, clu_v)
    for (src_h, dst_h) in ((e0_h, s_h), (e1_h, d_h)):
        pltpu.sync_copy(src_h.at[pl.ds(base, SLICE)], ein_v)

        def gbody(i, _):
            idx = ein_v[pl.ds(i * 16, 16)]
            out_v[pl.ds(i * 16, 16)] = plsc.load_gather(clu_v, [idx])
            return 0

        lax.fori_loop(0, SLICE // 16, gbody, 0)
        pltpu.sync_copy(out_v, dst_h.at[pl.ds(base, SLICE)])


@functools.partial(
    pl.kernel,
    out_type=[jax.ShapeDtypeStruct((NT * CAPU + 2048,), jnp.int32),
              jax.ShapeDtypeStruct((NT * CAPF * 16,), jnp.float32),
              jax.ShapeDtypeStruct((NT * 16,), jnp.int32)],
    mesh=_mesh,
    scratch_types=[
        pltpu.VMEM((CAP + 32,), jnp.int32),
        pltpu.VMEM((CAP + 32,), jnp.int32),
        pltpu.VMEM((CAP + 32,), jnp.int32),
        pltpu.VMEM((CAP + 32,), jnp.int32),
        pltpu.VMEM((W2,), jnp.int32),
        pltpu.VMEM((W2,), jnp.int32),
        pltpu.VMEM((W2,), jnp.int32),
        pltpu.VMEM((W2,), jnp.int32),
        pltpu.VMEM((2048,), jnp.int32),
        pltpu.VMEM((16, 128), jnp.float32),
        pltpu.VMEM((16, 128), jnp.float32),
        pltpu.VMEM((16,), jnp.int32),
        pltpu.VMEM((16,), jnp.int32),
        pltpu.VMEM((FST * 16,), jnp.float32),
        pltpu.VMEM((16,), jnp.int32),
        pltpu.SemaphoreType.DMA,
        pltpu.SemaphoreType.DMA,
        pltpu.SemaphoreType.DMA,
        pltpu.SemaphoreType.DMA,
    ],
    compiler_params=_params,
)
def _k2(s_h, d_h, f128_h, uq_h, fs_h, ud_h,
        pA, eA, pB, eB, winS0, winD0, winS1, winD1, hist,
        rowsA, rowsB, gidxA, gidxB, fstage, uvec,
        semA, semB, semW0, semW1):
    w = _wid()
    slo = w * TS
    shi = jnp.minimum(slo + TS, NCL)
    iot = _iota()

    winS = (winS0, winS1)
    winD = (winD0, winD1)
    semW = (semW0, semW1)

    def issue_win(widx, par):
        pltpu.async_copy(s_h.at[pl.ds(widx * W2, W2)], winS[par], semW[par])
        pltpu.async_copy(d_h.at[pl.ds(widx * W2, W2)], winD[par], semW[par])

    def wait_win(par):
        pltpu.make_async_copy(s_h.at[pl.ds(0, W2)], winS[par], semW[par]).wait()
        pltpu.make_async_copy(d_h.at[pl.ds(0, W2)], winD[par], semW[par]).wait()

    issue_win(0, 0)
    cnt = jnp.int32(0)
    for win in range(NWIN):
        par = win & 1
        wait_win(par)
        if win + 1 < NWIN:
            issue_win(win + 1, 1 - par)
        sbuf, dbuf = winS[par], winD[par]

        def fbody(i, c, sbuf=sbuf, dbuf=dbuf, win=win):
            sv = sbuf[pl.ds(i * 16, 16)]
            dv = dbuf[pl.ds(i * 16, 16)]
            m = (sv >= slo) & (sv < shi)
            p = (sv - slo) * 8192 + dv
            eid = (win * W2) + i * 16 + iot
            plsc.store_compressed(pA.at[pl.ds(c, 16)], p, mask=m)
            plsc.store_compressed(eA.at[pl.ds(c, 16)], eid, mask=m)
            npop = plsc.all_reduce_population_count(m)[0]
            return jnp.minimum(c + npop, CAP)

        cnt = lax.fori_loop(0, W2 // 16, fbody, cnt)

    sentv = jnp.zeros((16,), jnp.int32) + SENT
    zv = jnp.zeros((16,), jnp.int32)
    pA[pl.ds(cnt, 16)] = sentv
    eA[pl.ds(cnt, 16)] = zv
    cnt16 = (cnt + 15) & ~15
    pA[pl.ds(cnt16, 16)] = sentv
    eA[pl.ds(cnt16, 16)] = zv
    nv = ((cnt + 31) >> 5) << 1

    def radix_pass(pin, ein, pout, eout, shift, dmask):
        def zbody(j, _):
            hist[pl.ds(j * 16, 16)] = zv
            return 0
        lax.fori_loop(0, 128, zbody, 0)

        def hbody(i, _):
            dgt = (pin[pl.ds(i * 16, 16)] >> shift) & dmask
            c, last = plsc.scan_count(dgt)
            plsc.addupdate_scatter(hist.at[:], [dgt], c, mask=last)
            return 0
        lax.fori_loop(0, nv, hbody, 0)

        def sbody(j, carry):
            h = hist[pl.ds(j * 16, 16)]
            s = plsc.cumsum(h)
            hist[pl.ds(j * 16, 16)] = carry + s - h
            return carry + s[15]
        lax.fori_loop(0, 128, sbody, jnp.int32(0))

        def pbody(i, _):
            pv = pin[pl.ds(i * 16, 16)]
            ev = ein[pl.ds(i * 16, 16)]
            dgt = (pv >> shift) & dmask
            c, last = plsc.scan_count(dgt)
            b = plsc.load_gather(hist, [dgt])
            pos = b + c - 1
            plsc.store_scatter(pout.at[:], [pos], pv)
            plsc.store_scatter(eout.at[:], [pos], ev)
            plsc.addupdate_scatter(hist.at[:], [dgt], c, mask=last)
            return 0
        lax.fori_loop(0, nv, pbody, 0)

    radix_pass(pA, eA, pB, eB, 0, 1023)
    radix_pass(pB, eB, pA, eA, 10, 2047)

    rows = (rowsA, rowsB)
    gidx = (gidxA, gidxB)
    sems = (semA, semB)

    def issue_rows(chunk, par):
        gidx[par][...] = jnp.minimum(eA[pl.ds(chunk * 16, 16)] >> 3,
                                     E // 8 - 1)
        pltpu.async_copy(f128_h.at[gidx[par]], rows[par], sems[par])

    def wait_rows(par):
        pltpu.make_async_copy(f128_h.at[gidx[par]], rows[par],
                              sems[par]).wait()

    @pl.when(nv > 0)
    def _():
        issue_rows(0, 0)

    prev_perm = jnp.maximum(iot - 1, 0)

    def process_chunk(chunk, par, carry_p, ucnt, fbase):
        p = pA[pl.ds(chunk * 16, 16)]
        e = eA[pl.ds(chunk * 16, 16)]
        pprev = p.at[prev_perm].get(mode="promise_in_bounds")
        pprev = jnp.where(iot == 0, carry_p, pprev)
        validm = p != SENT
        first = (p != pprev) & validm
        firsti = jnp.where(first, 1, 0)
        r = ucnt + plsc.cumsum(firsti) - 1
        plsc.store_compressed(pB.at[pl.ds(ucnt, 16)], p, mask=first)
        ucnt_new = ucnt + plsc.all_reduce_population_count(first)[0]
        off = (e & 7) * 16
        sl = p >> 13
        dd = p & 8191
        selfm = (slo + sl) == dd
        validf = jnp.where(validm & (~selfm), 1.0, 0.0).astype(jnp.float32)
        rv = rows[par]
        for l in range(16):
            sub = rv[l, pl.ds(off[l], 16)] * validf[l]
            slot = (r[l] - fbase) * 16
            old = fstage[pl.ds(slot, 16)] * (1.0 - firsti[l].astype(jnp.float32))
            fstage[pl.ds(slot, 16)] = old + sub
        rl = r[15]
        do = (rl - fbase) >= (FLUSH + 16)

        @pl.when(do)
        def _():
            dst = pl.multiple_of((w * CAPF + fbase) * 16, 16)
            pltpu.sync_copy(fstage.at[pl.ds(0, FLUSH * 16)],
                            fs_h.at[pl.ds(dst, FLUSH * 16)])
            for j in range(FST - FLUSH):
                fstage[pl.ds(j * 16, 16)] = fstage[pl.ds((FLUSH + j) * 16, 16)]

        fbase_new = jnp.where(do, fbase + FLUSH, fbase)
        return p[15], ucnt_new, fbase_new

    def dbody(i, st):
        carry_p, ucnt, fbase = st
        c0 = 2 * i
        issue_rows(c0 + 1, 1)
        wait_rows(0)
        carry_p, ucnt, fbase = process_chunk(c0, 0, carry_p, ucnt, fbase)

        @pl.when(c0 + 2 < nv)
        def _():
            issue_rows(c0 + 2, 0)

        wait_rows(1)
        carry_p, ucnt, fbase = process_chunk(c0 + 1, 1, carry_p, ucnt, fbase)
        return carry_p, ucnt, fbase

    _, ucnt, fbase = lax.fori_loop(0, nv >> 1, dbody,
                                   (jnp.int32(-1), jnp.int32(0),
                                    jnp.int32(0)))

    for b, nrow in ((0, FLUSH), (1, FST - FLUSH)):
        @pl.when(b * FLUSH < ucnt - fbase)
        def _(b=b, nrow=nrow):
            dst = pl.multiple_of((w * CAPF + fbase + b * FLUSH) * 16, 16)
            pltpu.sync_copy(fstage.at[pl.ds(b * FLUSH * 16, nrow * 16)],
                            fs_h.at[pl.ds(dst, nrow * 16)])

    nu16 = (ucnt + 15) & ~15
    soff = jnp.int32(0)
    for k in range(10, -1, -1):
        sz = 16 << k
        bit = (nu16 >> (k + 4)) & 1

        @pl.when(bit == 1)
        def _(sz=sz, soff=soff):
            so = pl.multiple_of(soff, 16)
            dst = pl.multiple_of(w * CAPU + so, 16)
            pltpu.sync_copy(pB.at[pl.ds(so, sz)], uq_h.at[pl.ds(dst, sz)])

        soff = soff + bit * sz

    uvec[...] = jnp.where(iot == 0, ucnt, jnp.where(iot == 1, cnt, 0))
    pltpu.sync_copy(uvec, ud_h.at[pl.ds(w * 16, 16)])


@functools.partial(
    pl.kernel,
    out_type=[jax.ShapeDtypeStruct((E + 128,), jnp.int32),
              jax.ShapeDtypeStruct((E + 128,), jnp.int32),
              jax.ShapeDtypeStruct(((E + 256) * 16,), jnp.float32)],
    mesh=_mesh,
    scratch_types=[
        pltpu.VMEM((NT * 16,), jnp.int32),
        pltpu.VMEM((2048,), jnp.int32),
        pltpu.VMEM((128,), jnp.int32),
        pltpu.VMEM((128,), jnp.int32),
        pltpu.VMEM((128,), jnp.int32),
        pltpu.VMEM((160,), jnp.int32),
        pltpu.VMEM((16,), jnp.int32),
        pltpu.VMEM((16,), jnp.float32),
        pltpu.VMEM((FLUSH * 16,), jnp.float32),
        pltpu.VMEM((1024 * 16,), jnp.float32),
        pltpu.SemaphoreType.DMA,
        pltpu.SemaphoreType.DMA,
    ],
    compiler_params=_params,
)
def _k4(uq_h, fs_h, b16_h, srcp_h, dstp_h, ff_h,
        b16, uwin, idxb, valb, valb2, selfb, idxz, zrow, zblk, fbuf,
        sem0, sem1):
    w = _wid()
    iot = _iota()
    pltpu.sync_copy(b16_h, b16)
    bvec = b16[pl.ds(w * 16, 16)]
    base = bvec[0]
    U = bvec[1]
    total = bvec[2]

    zrow[...] = jnp.zeros((16,), jnp.float32)

    def zb(i, _):
        zblk[pl.ds(i * 16, 16)] = jnp.zeros((16,), jnp.float32)
        return 0
    lax.fori_loop(0, FLUSH, zb, 0)

    def abody(bi, scnt):
        c = bi & 15

        @pl.when(c == 0)
        def _():
            src = pl.multiple_of(w * CAPU + (bi >> 4) * 2048, 16)
            pltpu.sync_copy(uq_h.at[pl.ds(src, 2048)], uwin)

        jbase = bi * 128
        sc = scnt
        for q in range(8):
            j16 = jbase + q * 16
            p = uwin[pl.ds(c * 128 + q * 16, 16)]
            mval = (j16 + iot) < U
            sl = p >> 13
            dd = p & 8191
            s = w * TS + sl
            selfm = (s == dd) & mval
            srcv = jnp.where(selfm, -1, s)
            dstv = jnp.where(selfm, -1, dd)
            idxv = jnp.where(mval, base + j16 + iot, E + q * 16 + iot)
            idxb[pl.ds(q * 16, 16)] = idxv
            valb[pl.ds(q * 16, 16)] = srcv
            valb2[pl.ds(q * 16, 16)] = dstv
        pltpu.async_copy(valb, srcp_h.at[idxb], sem0).wait()
        pltpu.async_copy(valb2, dstp_h.at[idxb], sem1).wait()
        return sc

    def abody_g(bi, scnt):
        return lax.cond(bi * 128 < U, lambda s: abody(bi, s),
                        lambda s: s, scnt)

    scnt = lax.fori_loop(0, (CAP // 2048) * 16, abody_g, jnp.int32(0))

    def fcopy(soff, nrow):
        src = pl.multiple_of((w * CAPF + soff) * 16, 16)
        dst = pl.multiple_of((base + soff) * 16, 16)
        pltpu.sync_copy(fs_h.at[pl.ds(src, nrow * 16)],
                        fbuf.at[pl.ds(0, nrow * 16)])
        pltpu.sync_copy(fbuf.at[pl.ds(0, nrow * 16)],
                        ff_h.at[pl.ds(dst, nrow * 16)])

    def fb(i, _):
        fcopy(i * 1024, 1024)
        return 0
    lax.fori_loop(0, U >> 10, fb, 0)

    soff = (U >> 10) << 10
    for k in range(9, -1, -1):
        nrow = 1 << k
        bit = (U >> k) & 1

        @pl.when(bit == 1)
        def _(nrow=nrow, soff=soff):
            fcopy(soff, nrow)

        soff = soff + bit * nrow

    cpt = (E - total + NT - 1) >> 5

    def pfbody(b, _):
        @pl.when(b * FLUSH < cpt)
        def _():
            dst = pl.multiple_of((total + w * cpt + b * FLUSH) * 16, 16)
            pltpu.sync_copy(zblk, ff_h.at[pl.ds(dst, FLUSH * 16)])
        return 0
    lax.fori_loop(0, 40, pfbody, 0)

    negv = jnp.zeros((16,), jnp.int32) - 1
    hi = jnp.minimum(total + (w + 1) * cpt, E)

    def pdbody(cc, _):
        @pl.when(cc * 128 < cpt)
        def _():
            for q in range(8):
                qpos = total + w * cpt + cc * 128 + q * 16 + iot
                mval = qpos < hi
                idxb[pl.ds(q * 16, 16)] = jnp.where(mval, qpos,
                                                    E + q * 16 + iot)
                valb[pl.ds(q * 16, 16)] = negv
            pltpu.async_copy(valb, srcp_h.at[idxb], sem0).wait()
            pltpu.async_copy(valb, dstp_h.at[idxb], sem1).wait()
        return 0
    lax.fori_loop(0, 79, pdbody, 0)


def kernel(node_feat, edge_index, edge_feat):
    cluster = jax.random.randint(jax.random.key(42), (N,), 0, NCL)
    e0 = edge_index[0].astype(jnp.int32)
    e1 = edge_index[1].astype(jnp.int32)
    f128 = edge_feat.reshape(E // 8, 128)

    s_arr, d_arr = _k1(cluster.astype(jnp.int32), e0, e1)
    uq, fs, ud = _k2(s_arr, d_arr, f128)

    udm = ud.reshape(NT, 16)
    Uv = udm[:, 0].astype(jnp.int32)
    basev = jnp.concatenate([jnp.zeros((1,), jnp.int32),
                             jnp.cumsum(Uv)[:-1].astype(jnp.int32)])
    totalv = jnp.sum(Uv).astype(jnp.int32)
    b16 = jnp.zeros((NT, 16), jnp.int32)
    b16 = b16.at[:, 0].set(basev).at[:, 1].set(Uv).at[:, 2].set(totalv)
    srcp, dstp, ff = _k4(uq, fs, b16.reshape(-1))

    src = srcp[:E].astype(cluster.dtype)
    dst = dstp[:E].astype(cluster.dtype)
    new_edge_feat = ff[: E * 16].reshape(E, 16)

    old_nodes_idx = jnp.arange(N, dtype=cluster.dtype)
    new_dst_nodes = cluster + N
    inter_src = jnp.zeros(N * 2, dtype=cluster.dtype)
    inter_src = inter_src.at[0::2].set(old_nodes_idx).at[1::2].set(new_dst_nodes)
    inter_dst = jnp.zeros(N * 2, dtype=cluster.dtype)
    inter_dst = inter_dst.at[0::2].set(new_dst_nodes).at[1::2].set(old_nodes_idx)
    cluster_score = jnp.ones((NCL,), dtype=jnp.float32)
    return (src, dst, inter_src, inter_dst, cluster, new_edge_feat,
            cluster_score)
